# baseline (speedup 1.0000x reference)
import functools

import jax
import jax.numpy as jnp
from jax import lax
from jax.experimental import pallas as pl
from jax.experimental.pallas import tpu as pltpu
from jax.experimental.pallas import tpu_sc as plsc

_N = 10000
_E = 320000
_D = 128
_NEG = _E + _N
_NCAND = _NEG + _NEG // 16 + 1024
_CPAD = 352256
_SELSZ = _CPAD + 128
_MAPSZ = 100_000_512
_HALF = 50_000_000

_NW = 32
_EPAD = 330240
_RW = _EPAD // _NW
_CH = 240

_KC = 128
_CPW = _CPAD // 16 // _KC
_EPADM = 320128
_ECHUNKS = _EPADM // _KC


def _iota16():
    return lax.iota(jnp.int32, 16)


def _fill(ref, n, value):
    v = jnp.full((16,), value, jnp.int32)
    for i in range(n // 16):
        ref[pl.ds(16 * i, 16)] = v


_WRANGE = (_N * _N) // 32
_CCHUNKS = _CPAD // _KC


def _sc_sample_body(rowp_hbm, colp_hbm, cand_hbm, map_hbm, hit_hbm,
                    keyv, rowv, colv, hitv, posv, zerov, onev, sem):
    cid = lax.axis_index("c")
    sid = lax.axis_index("s")
    w32 = cid * 16 + sid
    wlo = w32 * _WRANGE
    trash0 = _N * _N + w32 * 16 + _iota16()
    _fill(zerov, _KC, 0)
    _fill(onev, _KC, 1)

    def zstep(k, carry):
        base = k * _KC
        pltpu.sync_copy(cand_hbm.at[pl.ds(base, _KC)], keyv)
        for i in range(_KC // 16):
            k16 = keyv[pl.ds(16 * i, 16)]
            inr = (k16 >= wlo) & (k16 < wlo + _WRANGE)
            posv[pl.ds(16 * i, 16)] = jnp.where(inr, k16, trash0)
        pltpu.async_copy(zerov, map_hbm.at[posv], sem).wait()
        return carry

    lax.fori_loop(0, _CCHUNKS, zstep, 0)
    plsc.subcore_barrier()

    def mstep(k, carry):
        base = k * _KC
        pltpu.sync_copy(rowp_hbm.at[pl.ds(base, _KC)], rowv)
        pltpu.sync_copy(colp_hbm.at[pl.ds(base, _KC)], colv)
        for i in range(_KC // 16):
            k16 = rowv[pl.ds(16 * i, 16)] * _N + colv[pl.ds(16 * i, 16)]
            inr = (k16 >= wlo) & (k16 < wlo + _WRANGE)
            posv[pl.ds(16 * i, 16)] = jnp.where(inr, k16, trash0)
        pltpu.async_copy(onev, map_hbm.at[posv], sem).wait()
        return carry

    lax.fori_loop(0, _ECHUNKS, mstep, 0)

    def lstep(k, carry):
        for i in range(_KC // 16):
            i16 = jnp.minimum(k * _KC + 16 * i + _iota16(), _N - 1)
            k16 = i16 * (_N + 1)
            inr = (k16 >= wlo) & (k16 < wlo + _WRANGE)
            posv[pl.ds(16 * i, 16)] = jnp.where(inr, k16, trash0)
        pltpu.async_copy(onev, map_hbm.at[posv], sem).wait()
        return carry

    lax.fori_loop(0, (_N + _KC - 1) // _KC, lstep, 0)
    plsc.subcore_barrier()

    lo = cid * _HALF
    cbase = sid * _CPW * _KC

    def hstep(k, carry):
        base = cbase + k * _KC
        pltpu.sync_copy(cand_hbm.at[pl.ds(base, _KC)], keyv)
        pltpu.async_copy(map_hbm.at[keyv], hitv, sem).wait()
        for i in range(_KC // 16):
            k16 = keyv[pl.ds(16 * i, 16)]
            inr = (k16 >= lo) & (k16 < lo + _HALF)
            h16 = hitv[pl.ds(16 * i, 16)]
            hitv[pl.ds(16 * i, 16)] = jnp.where(inr, h16, 0)
        pltpu.sync_copy(hitv, hit_hbm.at[cid, pl.ds(base, _KC)])
        return carry

    lax.fori_loop(0, _CPW, hstep, 0)


_sc_sample = functools.partial(
    pl.kernel,
    out_type=(
        jax.ShapeDtypeStruct((_MAPSZ,), jnp.int32),
        jax.ShapeDtypeStruct((2, _SELSZ), jnp.int32),
    ),
    mesh=plsc.VectorSubcoreMesh(core_axis_name="c", subcore_axis_name="s"),
    scratch_types=[
        pltpu.VMEM((_KC,), jnp.int32),
        pltpu.VMEM((_KC,), jnp.int32),
        pltpu.VMEM((_KC,), jnp.int32),
        pltpu.VMEM((_KC,), jnp.int32),
        pltpu.VMEM((_KC,), jnp.int32),
        pltpu.VMEM((_KC,), jnp.int32),
        pltpu.VMEM((_KC,), jnp.int32),
        pltpu.SemaphoreType.DMA,
    ],
)(_sc_sample_body)


_SEGLEN = _CPAD // 256


def _sc_compact_body(cand_hbm, hit_hbm, sel_hbm, cnt_hbm, part_hbm, work_hbm,
                     hv, h2v, cv, posv, valv, pv, qv, zv, sem):
    sid = lax.axis_index("s")
    sid16 = jnp.full((16,), sid, jnp.int32)
    cbase = sid * _CPW * _KC
    lane_gi0 = (sid * 16 + _iota16()) * _SEGLEN

    def cstep(k, acc):
        base = cbase + k * _KC
        pltpu.sync_copy(hit_hbm.at[0, pl.ds(base, _KC)], hv)
        pltpu.sync_copy(hit_hbm.at[1, pl.ds(base, _KC)], h2v)
        for i in range(_KC // 16):
            h16 = hv[pl.ds(16 * i, 16)] + h2v[pl.ds(16 * i, 16)]
            gi = lane_gi0 + (k * 8 + i)
            acc = acc + jnp.where((gi < _NCAND) & (h16 == 0), 1, 0)
        return acc

    acc = lax.fori_loop(0, _CPW, cstep, jnp.zeros((16,), jnp.int32))

    wbase = (lax.axis_index("c") * 16 + sid) * 48
    zv[...] = jnp.zeros((16,), jnp.int32)
    pltpu.sync_copy(zv, work_hbm.at[pl.ds(wbase, 16)])
    incl = acc
    for d in (1, 2, 4, 8):
        pv[...] = incl
        pltpu.sync_copy(pv, work_hbm.at[pl.ds(wbase + 16, 16)])
        qv[...] = wbase + 16 - d + _iota16()
        pltpu.async_copy(work_hbm.at[qv], zv, sem).wait()
        incl = incl + zv[...]
    pv[...] = incl
    pltpu.sync_copy(pv, work_hbm.at[pl.ds(wbase + 16, 16)])
    qv[...] = jnp.full((16,), wbase + 31, jnp.int32)
    pltpu.async_copy(work_hbm.at[qv], zv, sem).wait()
    pv[...] = zv[...]
    pltpu.sync_copy(pv, part_hbm.at[pl.ds(256 + sid * 16, 16)])
    plsc.subcore_barrier()

    base16 = jnp.zeros((16,), jnp.int32)
    total16 = jnp.zeros((16,), jnp.int32)
    for w in range(16):
        pltpu.sync_copy(part_hbm.at[pl.ds(256 + w * 16, 16)], pv)
        pw = pv[...]
        flag16 = jnp.maximum(jnp.minimum(sid16 - w, 1), 0)
        base16 = base16 + pw * flag16
        total16 = total16 + pw
    cnt16 = jnp.maximum(total16, 1)
    pv[...] = cnt16
    pltpu.sync_copy(pv, cnt_hbm)
    lane_base16 = base16 + incl - acc

    def sstep(k, running16):
        base = cbase + k * _KC
        pltpu.sync_copy(hit_hbm.at[0, pl.ds(base, _KC)], hv)
        pltpu.sync_copy(hit_hbm.at[1, pl.ds(base, _KC)], h2v)
        pltpu.sync_copy(cand_hbm.at[pl.ds(base, _KC)], cv)
        for i in range(_KC // 16):
            h16 = hv[pl.ds(16 * i, 16)] + h2v[pl.ds(16 * i, 16)]
            gi = lane_gi0 + (k * 8 + i)
            nh = jnp.where((gi < _NCAND) & (h16 == 0), 1, 0)
            pos = jnp.where(nh == 1, lane_base16 + running16,
                            _CPAD + 16 * i + _iota16())
            posv[pl.ds(16 * i, 16)] = pos
            valv[pl.ds(16 * i, 16)] = cv[pl.ds(16 * i, 16)]
            running16 = running16 + nh
        pltpu.async_copy(valv, sel_hbm.at[posv], sem).wait()
        return running16

    lax.fori_loop(0, _CPW, sstep, jnp.zeros((16,), jnp.int32))


_sc_compact = functools.partial(
    pl.kernel,
    out_type=(
        jax.ShapeDtypeStruct((_SELSZ,), jnp.int32),
        jax.ShapeDtypeStruct((16,), jnp.int32),
        jax.ShapeDtypeStruct((512,), jnp.int32),
        jax.ShapeDtypeStruct((1536,), jnp.int32),
    ),
    mesh=plsc.VectorSubcoreMesh(core_axis_name="c", subcore_axis_name="s"),
    scratch_types=[
        pltpu.VMEM((_KC,), jnp.int32),
        pltpu.VMEM((_KC,), jnp.int32),
        pltpu.VMEM((_KC,), jnp.int32),
        pltpu.VMEM((_KC,), jnp.int32),
        pltpu.VMEM((_KC,), jnp.int32),
        pltpu.VMEM((16,), jnp.int32),
        pltpu.VMEM((16,), jnp.int32),
        pltpu.VMEM((16,), jnp.int32),
        pltpu.SemaphoreType.DMA,
    ],
)(_sc_compact_body)


def _sc_gather_body(sel_hbm, cnt_hbm, x_hbm, xi_hbm, xj_hbm,
                    cntv, mmv, keyv, av, bv, rows_v, sem):
    wid = lax.axis_index("s") * 2 + lax.axis_index("c")
    base = wid * _RW
    pltpu.sync_copy(cnt_hbm, cntv)
    cnt16 = cntv[...]

    def step(k, carry):
        off = base + k * _CH
        for i in range(_CH // 16):
            m16 = off + 16 * i + _iota16()
            mmv[pl.ds(16 * i, 16)] = lax.rem(m16, cnt16)
        pltpu.async_copy(sel_hbm.at[mmv], keyv, sem).wait()
        for i in range(_CH // 16):
            k16 = keyv[pl.ds(16 * i, 16)]
            av[pl.ds(16 * i, 16)] = lax.div(k16, _N)
            bv[pl.ds(16 * i, 16)] = lax.rem(k16, _N)
        pltpu.async_copy(x_hbm.at[bv], rows_v, sem).wait()
        pltpu.sync_copy(rows_v, xi_hbm.at[pl.ds(off, _CH)])
        pltpu.async_copy(x_hbm.at[av], rows_v, sem).wait()
        pltpu.sync_copy(rows_v, xj_hbm.at[pl.ds(off, _CH)])
        return carry

    lax.fori_loop(0, _RW // _CH, step, 0)


_sc_gather = functools.partial(
    pl.kernel,
    out_type=(
        jax.ShapeDtypeStruct((_EPAD, _D), jnp.float32),
        jax.ShapeDtypeStruct((_EPAD, _D), jnp.float32),
    ),
    mesh=plsc.VectorSubcoreMesh(core_axis_name="c", subcore_axis_name="s"),
    scratch_types=[
        pltpu.VMEM((16,), jnp.int32),
        pltpu.VMEM((_CH,), jnp.int32),
        pltpu.VMEM((_CH,), jnp.int32),
        pltpu.VMEM((_CH,), jnp.int32),
        pltpu.VMEM((_CH,), jnp.int32),
        pltpu.VMEM((_CH, _D), jnp.float32),
        pltpu.SemaphoreType.DMA,
    ],
)(_sc_gather_body)


_SB = 512


def _score_body(xi_ref, xj_ref, o_ref):
    v = xi_ref[...] * xj_ref[...]
    acc = v[:, 0:8]
    for k in range(1, 16):
        acc = acc + v[:, 8 * k:8 * k + 8]
    t = acc[:, 0:4] + acc[:, 4:8]
    t = t[:, 0:2] + t[:, 2:4]
    o_ref[...] = t[:, 0] + t[:, 1]


def _tc_scores(xi, xj):
    grid = _EPAD // _SB
    return pl.pallas_call(
        _score_body,
        grid=(grid,),
        in_specs=[
            pl.BlockSpec((_SB, _D), lambda i: (i, 0)),
            pl.BlockSpec((_SB, _D), lambda i: (i, 0)),
        ],
        out_specs=pl.BlockSpec((_SB,), lambda i: (i,)),
        out_shape=jax.ShapeDtypeStruct((_EPAD,), jnp.float32),
    )(xi, xj)


_TP = 1024


def _rank_body(t_ref, s_ref, o_ref):
    pid = pl.program_id(0)
    tv = t_ref[...]

    @pl.when(pid == 0)
    def _():
        ik = lax.broadcasted_iota(jnp.int32, (_TP, _TP), 0)
        ij = lax.broadcasted_iota(jnp.int32, (_TP, _TP), 1)
        tri = jnp.where(ik < ij, 1, 0)
        tie = jnp.where(tv[:, None] == tv[None, :], tri, 0)
        o_ref[...] = jnp.sum(tie, axis=0)

    sv = s_ref[...]
    gidx = pid * _SB + lax.broadcasted_iota(jnp.int32, (_SB,), 0)
    vi = jnp.where(gidx < _NEG, 1, 0)
    cmp = jnp.where(sv[:, None] > tv[None, :], vi[:, None], 0)
    o_ref[...] = o_ref[...] + jnp.sum(cmp, axis=0)


def _tc_ranks(targets, scores):
    grid = _EPAD // _SB
    return pl.pallas_call(
        _rank_body,
        grid=(grid,),
        in_specs=[
            pl.BlockSpec((_TP,), lambda i: (0,)),
            pl.BlockSpec((_SB,), lambda i: (i,)),
        ],
        out_specs=pl.BlockSpec((_TP,), lambda i: (0,)),
        out_shape=jax.ShapeDtypeStruct((_TP,), jnp.int32),
    )(targets, scores)


def _sc_final_body(rank_hbm, sel_hbm, cnt_hbm, out_hbm,
                   cntv, rv, mmv, kv, av, bv, sem):
    wid = lax.axis_index("s") * 2 + lax.axis_index("c")
    base = wid * (_TP // _NW)
    pltpu.sync_copy(cnt_hbm, cntv)
    cnt16 = cntv[...]
    pltpu.sync_copy(rank_hbm.at[pl.ds(base, 32)], rv)
    for i in range(2):
        r16 = rv[pl.ds(16 * i, 16)]
        mmv[pl.ds(16 * i, 16)] = lax.rem(r16, cnt16)
    pltpu.async_copy(sel_hbm.at[mmv], kv, sem).wait()
    for i in range(2):
        k16 = kv[pl.ds(16 * i, 16)]
        av[pl.ds(16 * i, 16)] = lax.div(k16, _N)
        bv[pl.ds(16 * i, 16)] = lax.rem(k16, _N)
    pltpu.sync_copy(av, out_hbm.at[0, pl.ds(base, 32)])
    pltpu.sync_copy(bv, out_hbm.at[1, pl.ds(base, 32)])


_sc_final = functools.partial(
    pl.kernel,
    out_type=jax.ShapeDtypeStruct((2, _TP), jnp.int32),
    mesh=plsc.VectorSubcoreMesh(core_axis_name="c", subcore_axis_name="s"),
    scratch_types=[
        pltpu.VMEM((16,), jnp.int32),
        pltpu.VMEM((32,), jnp.int32),
        pltpu.VMEM((32,), jnp.int32),
        pltpu.VMEM((32,), jnp.int32),
        pltpu.VMEM((32,), jnp.int32),
        pltpu.VMEM((32,), jnp.int32),
        pltpu.SemaphoreType.DMA,
    ],
)(_sc_final_body)


def kernel(x, edge_index, r_scaling_1, r_bias_1, r_scaling_2, r_bias_2,
           r_scaling_3, r_bias_3, r_scaling_4, r_bias_4, r_scaling_5,
           r_bias_5):
    zpad = jnp.zeros((_EPADM - _E,), jnp.int32)
    rowp = jnp.concatenate([edge_index[0], zpad])
    colp = jnp.concatenate([edge_index[1], zpad])

    key = jax.random.key(42)
    cand = jax.random.randint(key, (_NCAND,), 0, _N * _N)
    cand_pad = jnp.concatenate(
        [cand, jnp.zeros((_CPAD - _NCAND,), cand.dtype)]).astype(jnp.int32)
    cand_t = cand_pad.reshape(16, 16, _SEGLEN).transpose(0, 2, 1).reshape(-1)

    _map_scratch, hit = _sc_sample(rowp, colp, cand_t)
    sel, cnt, _part, _work = _sc_compact(cand_t, hit)
    xi, xj = _sc_gather(sel, cnt, x)

    s = _tc_scores(xi, xj)
    s = r_scaling_1 * jax.nn.elu(s) + r_bias_1
    s = r_scaling_2 * jax.nn.elu(s) + r_bias_2
    s = r_scaling_3 * jax.nn.elu(s) + r_bias_3
    s = r_scaling_4 * jax.nn.elu(s) + r_bias_4
    s = r_scaling_5 * jax.nn.elu(s) + r_bias_5

    ranks = _tc_ranks(s[:_TP], s)
    out = _sc_final(ranks, sel, cnt)
    return out[:, :1000]


# speedup vs baseline: 118.8891x; 118.8891x over previous
import functools

import jax
import jax.numpy as jnp
from jax import lax
from jax.experimental import pallas as pl
from jax.experimental.pallas import tpu as pltpu
from jax.experimental.pallas import tpu_sc as plsc

_N = 10000
_E = 320000
_D = 128
_NEG = _E + _N
_NCAND = _NEG + _NEG // 16 + 1024
_CPAD = 352256
_SELSZ = _CPAD + 128
_MAPSZ = 1_600_004_096
_HALF = 50_000_000

_NW = 32
_EPAD = 330240
_RW = _EPAD // _NW
_CH = 240

_KC = 128
_CPW = _CPAD // 16 // _KC
_EPADM = 320128
_ECHUNKS = _EPADM // _KC


def _iota16():
    return lax.iota(jnp.int32, 16)


def _fill(ref, n, value):
    v = jnp.full((16,), value, jnp.int32)
    for i in range(n // 16):
        ref[pl.ds(16 * i, 16)] = v


_STRIDE = 16
_TRASH0 = _N * _N * _STRIDE


def _sc_sample_body(rowp_hbm, colp_hbm, cand_hbm, map_hbm, hit_hbm,
                    keyv, rowv, colv, hitv, posv, zerov, onev, sem):
    cid = lax.axis_index("c")
    sid = lax.axis_index("s")
    w32 = cid * 16 + sid
    lo = cid * _HALF
    _fill(zerov, _KC, 0)
    _fill(onev, _KC, 1)
    cbase = sid * _CPW * _KC

    def route(k16, i):
        inr = (k16 >= lo) & (k16 < lo + _HALF)
        return jnp.where(inr, k16 * _STRIDE,
                         _TRASH0 + w32 * _KC + 16 * i + _iota16())

    def zstep(k, carry):
        base = cbase + k * _KC
        pltpu.sync_copy(cand_hbm.at[pl.ds(base, _KC)], keyv)
        for i in range(_KC // 16):
            posv[pl.ds(16 * i, 16)] = route(keyv[pl.ds(16 * i, 16)], i)
        pltpu.async_copy(zerov, map_hbm.at[posv], sem).wait()
        return carry

    lax.fori_loop(0, _CPW, zstep, 0)
    plsc.subcore_barrier()

    def mstep(k, carry):
        cidx = jnp.minimum(sid + k * 16, _ECHUNKS - 1)
        base = cidx * _KC
        pltpu.sync_copy(rowp_hbm.at[pl.ds(base, _KC)], rowv)
        pltpu.sync_copy(colp_hbm.at[pl.ds(base, _KC)], colv)
        for i in range(_KC // 16):
            k16 = rowv[pl.ds(16 * i, 16)] * _N + colv[pl.ds(16 * i, 16)]
            posv[pl.ds(16 * i, 16)] = route(k16, i)
        pltpu.async_copy(onev, map_hbm.at[posv], sem).wait()
        return carry

    lax.fori_loop(0, (_ECHUNKS + 15) // 16, mstep, 0)

    def lstep(k, carry):
        cidx = jnp.minimum(sid + k * 16, (_N + _KC - 1) // _KC - 1)
        for i in range(_KC // 16):
            i16 = jnp.minimum(cidx * _KC + 16 * i + _iota16(), _N - 1)
            posv[pl.ds(16 * i, 16)] = route(i16 * (_N + 1), i)
        pltpu.async_copy(onev, map_hbm.at[posv], sem).wait()
        return carry

    lax.fori_loop(0, 5, lstep, 0)
    plsc.subcore_barrier()

    def hstep(k, carry):
        base = cbase + k * _KC
        pltpu.sync_copy(cand_hbm.at[pl.ds(base, _KC)], keyv)
        for i in range(_KC // 16):
            posv[pl.ds(16 * i, 16)] = keyv[pl.ds(16 * i, 16)] * _STRIDE
        pltpu.async_copy(map_hbm.at[posv], hitv, sem).wait()
        for i in range(_KC // 16):
            k16 = keyv[pl.ds(16 * i, 16)]
            inr = (k16 >= lo) & (k16 < lo + _HALF)
            h16 = hitv[pl.ds(16 * i, 16)]
            hitv[pl.ds(16 * i, 16)] = jnp.where(inr, h16, 0)
        pltpu.sync_copy(hitv, hit_hbm.at[cid, pl.ds(base, _KC)])
        return carry

    lax.fori_loop(0, _CPW, hstep, 0)


_sc_sample = functools.partial(
    pl.kernel,
    out_type=(
        jax.ShapeDtypeStruct((_MAPSZ,), jnp.int32),
        jax.ShapeDtypeStruct((2, _SELSZ), jnp.int32),
    ),
    mesh=plsc.VectorSubcoreMesh(core_axis_name="c", subcore_axis_name="s"),
    scratch_types=[
        pltpu.VMEM((_KC,), jnp.int32),
        pltpu.VMEM((_KC,), jnp.int32),
        pltpu.VMEM((_KC,), jnp.int32),
        pltpu.VMEM((_KC,), jnp.int32),
        pltpu.VMEM((_KC,), jnp.int32),
        pltpu.VMEM((_KC,), jnp.int32),
        pltpu.VMEM((_KC,), jnp.int32),
        pltpu.SemaphoreType.DMA,
    ],
)(_sc_sample_body)


_SEGLEN = _CPAD // 256


def _sc_compact_body(cand_hbm, hit_hbm, sel_hbm, cnt_hbm, part_hbm, work_hbm,
                     hv, h2v, cv, posv, valv, pv, qv, zv, sem):
    sid = lax.axis_index("s")
    sid16 = jnp.full((16,), sid, jnp.int32)
    cbase = sid * _CPW * _KC
    lane_gi0 = (sid * 16 + _iota16()) * _SEGLEN

    def cstep(k, acc):
        base = cbase + k * _KC
        pltpu.sync_copy(hit_hbm.at[0, pl.ds(base, _KC)], hv)
        pltpu.sync_copy(hit_hbm.at[1, pl.ds(base, _KC)], h2v)
        for i in range(_KC // 16):
            h16 = hv[pl.ds(16 * i, 16)] + h2v[pl.ds(16 * i, 16)]
            gi = lane_gi0 + (k * 8 + i)
            acc = acc + jnp.where((gi < _NCAND) & (h16 == 0), 1, 0)
        return acc

    acc = lax.fori_loop(0, _CPW, cstep, jnp.zeros((16,), jnp.int32))

    wbase = (lax.axis_index("c") * 16 + sid) * 48
    zv[...] = jnp.zeros((16,), jnp.int32)
    pltpu.sync_copy(zv, work_hbm.at[pl.ds(wbase, 16)])
    incl = acc
    for d in (1, 2, 4, 8):
        pv[...] = incl
        pltpu.sync_copy(pv, work_hbm.at[pl.ds(wbase + 16, 16)])
        qv[...] = wbase + 16 - d + _iota16()
        pltpu.async_copy(work_hbm.at[qv], zv, sem).wait()
        incl = incl + zv[...]
    pv[...] = incl
    pltpu.sync_copy(pv, work_hbm.at[pl.ds(wbase + 16, 16)])
    qv[...] = jnp.full((16,), wbase + 31, jnp.int32)
    pltpu.async_copy(work_hbm.at[qv], zv, sem).wait()
    pv[...] = zv[...]
    pltpu.sync_copy(pv, part_hbm.at[pl.ds(256 + sid * 16, 16)])
    plsc.subcore_barrier()

    base16 = jnp.zeros((16,), jnp.int32)
    total16 = jnp.zeros((16,), jnp.int32)
    for w in range(16):
        pltpu.sync_copy(part_hbm.at[pl.ds(256 + w * 16, 16)], pv)
        pw = pv[...]
        flag16 = jnp.maximum(jnp.minimum(sid16 - w, 1), 0)
        base16 = base16 + pw * flag16
        total16 = total16 + pw
    cnt16 = jnp.maximum(total16, 1)
    pv[...] = cnt16
    pltpu.sync_copy(pv, cnt_hbm)
    lane_base16 = base16 + incl - acc

    def sstep(k, running16):
        base = cbase + k * _KC
        pltpu.sync_copy(hit_hbm.at[0, pl.ds(base, _KC)], hv)
        pltpu.sync_copy(hit_hbm.at[1, pl.ds(base, _KC)], h2v)
        pltpu.sync_copy(cand_hbm.at[pl.ds(base, _KC)], cv)
        for i in range(_KC // 16):
            h16 = hv[pl.ds(16 * i, 16)] + h2v[pl.ds(16 * i, 16)]
            gi = lane_gi0 + (k * 8 + i)
            nh = jnp.where((gi < _NCAND) & (h16 == 0), 1, 0)
            pos = jnp.where(nh == 1, lane_base16 + running16,
                            _CPAD + 16 * i + _iota16())
            posv[pl.ds(16 * i, 16)] = pos
            valv[pl.ds(16 * i, 16)] = cv[pl.ds(16 * i, 16)]
            running16 = running16 + nh
        pltpu.async_copy(valv, sel_hbm.at[posv], sem).wait()
        return running16

    lax.fori_loop(0, _CPW, sstep, jnp.zeros((16,), jnp.int32))


_sc_compact = functools.partial(
    pl.kernel,
    out_type=(
        jax.ShapeDtypeStruct((_SELSZ,), jnp.int32),
        jax.ShapeDtypeStruct((16,), jnp.int32),
        jax.ShapeDtypeStruct((512,), jnp.int32),
        jax.ShapeDtypeStruct((1536,), jnp.int32),
    ),
    mesh=plsc.VectorSubcoreMesh(core_axis_name="c", subcore_axis_name="s"),
    scratch_types=[
        pltpu.VMEM((_KC,), jnp.int32),
        pltpu.VMEM((_KC,), jnp.int32),
        pltpu.VMEM((_KC,), jnp.int32),
        pltpu.VMEM((_KC,), jnp.int32),
        pltpu.VMEM((_KC,), jnp.int32),
        pltpu.VMEM((16,), jnp.int32),
        pltpu.VMEM((16,), jnp.int32),
        pltpu.VMEM((16,), jnp.int32),
        pltpu.SemaphoreType.DMA,
    ],
)(_sc_compact_body)


def _sc_gather_body(sel_hbm, cnt_hbm, x_hbm, xi_hbm, xj_hbm,
                    cntv, mmv, keyv, av, bv, rows_v, sem):
    wid = lax.axis_index("s") * 2 + lax.axis_index("c")
    base = wid * _RW
    pltpu.sync_copy(cnt_hbm, cntv)
    cnt16 = cntv[...]

    def step(k, carry):
        off = base + k * _CH
        for i in range(_CH // 16):
            m16 = off + 16 * i + _iota16()
            mmv[pl.ds(16 * i, 16)] = lax.rem(m16, cnt16)
        pltpu.async_copy(sel_hbm.at[mmv], keyv, sem).wait()
        for i in range(_CH // 16):
            k16 = keyv[pl.ds(16 * i, 16)]
            av[pl.ds(16 * i, 16)] = lax.div(k16, _N)
            bv[pl.ds(16 * i, 16)] = lax.rem(k16, _N)
        pltpu.async_copy(x_hbm.at[bv], rows_v, sem).wait()
        pltpu.sync_copy(rows_v, xi_hbm.at[pl.ds(off, _CH)])
        pltpu.async_copy(x_hbm.at[av], rows_v, sem).wait()
        pltpu.sync_copy(rows_v, xj_hbm.at[pl.ds(off, _CH)])
        return carry

    lax.fori_loop(0, _RW // _CH, step, 0)


_sc_gather = functools.partial(
    pl.kernel,
    out_type=(
        jax.ShapeDtypeStruct((_EPAD, _D), jnp.float32),
        jax.ShapeDtypeStruct((_EPAD, _D), jnp.float32),
    ),
    mesh=plsc.VectorSubcoreMesh(core_axis_name="c", subcore_axis_name="s"),
    scratch_types=[
        pltpu.VMEM((16,), jnp.int32),
        pltpu.VMEM((_CH,), jnp.int32),
        pltpu.VMEM((_CH,), jnp.int32),
        pltpu.VMEM((_CH,), jnp.int32),
        pltpu.VMEM((_CH,), jnp.int32),
        pltpu.VMEM((_CH, _D), jnp.float32),
        pltpu.SemaphoreType.DMA,
    ],
)(_sc_gather_body)


_SB = 512


def _score_body(xi_ref, xj_ref, o_ref):
    v = xi_ref[...] * xj_ref[...]
    acc = v[:, 0:8]
    for k in range(1, 16):
        acc = acc + v[:, 8 * k:8 * k + 8]
    t = acc[:, 0:4] + acc[:, 4:8]
    t = t[:, 0:2] + t[:, 2:4]
    o_ref[...] = t[:, 0] + t[:, 1]


def _tc_scores(xi, xj):
    grid = _EPAD // _SB
    return pl.pallas_call(
        _score_body,
        grid=(grid,),
        in_specs=[
            pl.BlockSpec((_SB, _D), lambda i: (i, 0)),
            pl.BlockSpec((_SB, _D), lambda i: (i, 0)),
        ],
        out_specs=pl.BlockSpec((_SB,), lambda i: (i,)),
        out_shape=jax.ShapeDtypeStruct((_EPAD,), jnp.float32),
    )(xi, xj)


_TP = 1024


def _rank_body(t_ref, s_ref, o_ref):
    pid = pl.program_id(0)
    tv = t_ref[...]

    @pl.when(pid == 0)
    def _():
        ik = lax.broadcasted_iota(jnp.int32, (_TP, _TP), 0)
        ij = lax.broadcasted_iota(jnp.int32, (_TP, _TP), 1)
        tri = jnp.where(ik < ij, 1, 0)
        tie = jnp.where(tv[:, None] == tv[None, :], tri, 0)
        o_ref[...] = jnp.sum(tie, axis=0)

    sv = s_ref[...]
    gidx = pid * _SB + lax.broadcasted_iota(jnp.int32, (_SB,), 0)
    vi = jnp.where(gidx < _NEG, 1, 0)
    cmp = jnp.where(sv[:, None] > tv[None, :], vi[:, None], 0)
    o_ref[...] = o_ref[...] + jnp.sum(cmp, axis=0)


def _tc_ranks(targets, scores):
    grid = _EPAD // _SB
    return pl.pallas_call(
        _rank_body,
        grid=(grid,),
        in_specs=[
            pl.BlockSpec((_TP,), lambda i: (0,)),
            pl.BlockSpec((_SB,), lambda i: (i,)),
        ],
        out_specs=pl.BlockSpec((_TP,), lambda i: (0,)),
        out_shape=jax.ShapeDtypeStruct((_TP,), jnp.int32),
    )(targets, scores)


def _sc_final_body(rank_hbm, sel_hbm, cnt_hbm, out_hbm,
                   cntv, rv, mmv, kv, av, bv, sem):
    wid = lax.axis_index("s") * 2 + lax.axis_index("c")
    base = wid * (_TP // _NW)
    pltpu.sync_copy(cnt_hbm, cntv)
    cnt16 = cntv[...]
    pltpu.sync_copy(rank_hbm.at[pl.ds(base, 32)], rv)
    for i in range(2):
        r16 = rv[pl.ds(16 * i, 16)]
        mmv[pl.ds(16 * i, 16)] = lax.rem(r16, cnt16)
    pltpu.async_copy(sel_hbm.at[mmv], kv, sem).wait()
    for i in range(2):
        k16 = kv[pl.ds(16 * i, 16)]
        av[pl.ds(16 * i, 16)] = lax.div(k16, _N)
        bv[pl.ds(16 * i, 16)] = lax.rem(k16, _N)
    pltpu.sync_copy(av, out_hbm.at[0, pl.ds(base, 32)])
    pltpu.sync_copy(bv, out_hbm.at[1, pl.ds(base, 32)])


_sc_final = functools.partial(
    pl.kernel,
    out_type=jax.ShapeDtypeStruct((2, _TP), jnp.int32),
    mesh=plsc.VectorSubcoreMesh(core_axis_name="c", subcore_axis_name="s"),
    scratch_types=[
        pltpu.VMEM((16,), jnp.int32),
        pltpu.VMEM((32,), jnp.int32),
        pltpu.VMEM((32,), jnp.int32),
        pltpu.VMEM((32,), jnp.int32),
        pltpu.VMEM((32,), jnp.int32),
        pltpu.VMEM((32,), jnp.int32),
        pltpu.SemaphoreType.DMA,
    ],
)(_sc_final_body)


def kernel(x, edge_index, r_scaling_1, r_bias_1, r_scaling_2, r_bias_2,
           r_scaling_3, r_bias_3, r_scaling_4, r_bias_4, r_scaling_5,
           r_bias_5):
    zpad = jnp.zeros((_EPADM - _E,), jnp.int32)
    rowp = jnp.concatenate([edge_index[0], zpad])
    colp = jnp.concatenate([edge_index[1], zpad])

    key = jax.random.key(42)
    cand = jax.random.randint(key, (_NCAND,), 0, _N * _N)
    cand_pad = jnp.concatenate(
        [cand, jnp.zeros((_CPAD - _NCAND,), cand.dtype)]).astype(jnp.int32)
    cand_t = cand_pad.reshape(16, 16, _SEGLEN).transpose(0, 2, 1).reshape(-1)

    _map_scratch, hit = _sc_sample(rowp, colp, cand_t)
    sel, cnt, _part, _work = _sc_compact(cand_t, hit)
    xi, xj = _sc_gather(sel, cnt, x)

    s = _tc_scores(xi, xj)
    s = r_scaling_1 * jax.nn.elu(s) + r_bias_1
    s = r_scaling_2 * jax.nn.elu(s) + r_bias_2
    s = r_scaling_3 * jax.nn.elu(s) + r_bias_3
    s = r_scaling_4 * jax.nn.elu(s) + r_bias_4
    s = r_scaling_5 * jax.nn.elu(s) + r_bias_5

    ranks = _tc_ranks(s[:_TP], s)
    out = _sc_final(ranks, sel, cnt)
    return out[:, :1000]
